# Initial kernel scaffold; baseline (speedup 1.0000x reference)
import jax
import jax.numpy as jnp
from jax.experimental import pallas as pl


def kernel(x, edge_index, edge_attr, mW0, mb0, mW1, mb1, mW2, mb2, mg, mbeta, uW0, ub0, uW1, ub1, uW2, ub2, ug, ubeta):
    raise NotImplementedError("write your pallas kernel here")


# speedup vs baseline: 3.9717x; 3.9717x over previous
import functools

import jax
import jax.numpy as jnp
from jax import lax
from jax.experimental import pallas as pl
from jax.experimental.pallas import tpu as pltpu
from jax.experimental.pallas import tpu_sc as plsc


def _make_sc_gather(N, E, D, ch):
    info = plsc.get_sparse_core_info()
    NC, NS = info.num_cores, info.num_subcores
    NW = NC * NS
    epw = E // NW
    nchunk = epw // ch
    mesh = plsc.VectorSubcoreMesh(core_axis_name="c", subcore_axis_name="s")

    @functools.partial(
        pl.kernel,
        out_type=[jax.ShapeDtypeStruct((E, D), jnp.float32),
                  jax.ShapeDtypeStruct((E, D), jnp.float32)],
        mesh=mesh,
        scratch_types=[pltpu.VMEM((ch,), jnp.int32),
                       pltpu.VMEM((ch,), jnp.int32),
                       pltpu.VMEM((ch, D), jnp.float32),
                       pltpu.VMEM((ch, D), jnp.float32),
                       pltpu.SemaphoreType.DMA,
                       pltpu.SemaphoreType.DMA],
    )
    def gather_k(x_hbm, src_hbm, dst_hbm, xj_hbm, xi_hbm,
                 idx_a, idx_b, rows_a, rows_b, sem_a, sem_b):
        wid = lax.axis_index("s") * NC + lax.axis_index("c")
        base = wid * epw

        def body(i, _):
            off = base + i * ch
            pltpu.sync_copy(src_hbm.at[pl.ds(off, ch)], idx_a)
            pltpu.sync_copy(dst_hbm.at[pl.ds(off, ch)], idx_b)
            ca = pltpu.async_copy(x_hbm.at[idx_a], rows_a, sem_a)
            cb = pltpu.async_copy(x_hbm.at[idx_b], rows_b, sem_b)
            ca.wait()
            cb.wait()
            pltpu.sync_copy(rows_a, xj_hbm.at[pl.ds(off, ch)])
            pltpu.sync_copy(rows_b, xi_hbm.at[pl.ds(off, ch)])
            return 0

        lax.fori_loop(0, nchunk, body, 0)

    return gather_k


def _make_sc_scatter(Np, E, D, ch):
    info = plsc.get_sparse_core_info()
    NC, NS = info.num_cores, info.num_subcores
    epc = E // NC
    ept = epc // NS
    nchunk = ept // ch
    rpt = Np // NS
    mesh = plsc.VectorSubcoreMesh(core_axis_name="c", subcore_axis_name="s")

    @functools.partial(
        pl.kernel,
        out_type=jax.ShapeDtypeStruct((NC, Np, D), jnp.float32),
        mesh=mesh,
        scratch_types=[pltpu.VMEM((ch,), jnp.int32),
                       pltpu.VMEM((ch, D), jnp.float32),
                       pltpu.VMEM_SHARED((Np, D), jnp.float32)],
    )
    def scatter_k(msg_hbm, dst_hbm, zeros_hbm, out_hbm, idx_v, rows_v, acc_sh):
        cid = lax.axis_index("c")
        sid = lax.axis_index("s")
        pltpu.sync_copy(zeros_hbm.at[pl.ds(sid * rpt, rpt)],
                        acc_sh.at[pl.ds(sid * rpt, rpt)])
        plsc.subcore_barrier()

        base = cid * epc + sid * ept

        def body(i, _):
            off = base + i * ch
            pltpu.sync_copy(dst_hbm.at[pl.ds(off, ch)], idx_v)
            pltpu.sync_copy(msg_hbm.at[pl.ds(off, ch)], rows_v)
            pltpu.sync_copy(rows_v, acc_sh.at[idx_v], add=True)
            return 0

        lax.fori_loop(0, nchunk, body, 0)
        plsc.subcore_barrier()
        pltpu.sync_copy(acc_sh.at[pl.ds(sid * rpt, rpt)],
                        out_hbm.at[cid, pl.ds(sid * rpt, rpt)])

    return scatter_k


def _edge_mlp_body(xi_ref, xj_ref, ea_ref, W0_ref, b0_ref, W1_ref, b1_ref,
                   W2_ref, b2_ref, g_ref, beta_ref, new_ref, eout_ref):
    D = xi_ref.shape[1]
    xi = xi_ref[...]
    xj = xj_ref[...]
    ea = ea_ref[...]
    h = (jnp.dot(xi, W0_ref[0:D, :], preferred_element_type=jnp.float32)
         + jnp.dot(xj, W0_ref[D:2 * D, :], preferred_element_type=jnp.float32)
         + jnp.dot(ea, W0_ref[2 * D:3 * D, :], preferred_element_type=jnp.float32)
         + b0_ref[...])
    h = jnp.maximum(h, 0.0)
    h = jnp.maximum(jnp.dot(h, W1_ref[...], preferred_element_type=jnp.float32)
                    + b1_ref[...], 0.0)
    h = jnp.dot(h, W2_ref[...], preferred_element_type=jnp.float32) + b2_ref[...]
    mu = jnp.mean(h, axis=-1, keepdims=True)
    hc = h - mu
    var = jnp.mean(hc * hc, axis=-1, keepdims=True)
    hn = hc * lax.rsqrt(var + 1e-5)
    new = hn * g_ref[...] + beta_ref[...]
    new_ref[...] = new
    eout_ref[...] = new + ea


def _tc_edge_mlp(xi, xj, ea, W0, b0, W1, b1, W2, b2, g, beta, block):
    E, D = ea.shape
    grid = (E // block,)
    row_spec = pl.BlockSpec((block, D), lambda i: (i, 0))
    full = lambda shape: pl.BlockSpec(shape, lambda i: (0,) * len(shape))
    return pl.pallas_call(
        _edge_mlp_body,
        grid=grid,
        in_specs=[row_spec, row_spec, row_spec,
                  full((3 * D, D)), full((1, D)),
                  full((D, D)), full((1, D)),
                  full((D, D)), full((1, D)),
                  full((1, D)), full((1, D))],
        out_specs=[row_spec, row_spec],
        out_shape=[jax.ShapeDtypeStruct((E, D), jnp.float32),
                   jax.ShapeDtypeStruct((E, D), jnp.float32)],
    )(xi, xj, ea, W0, b0.reshape(1, D), W1, b1.reshape(1, D),
      W2, b2.reshape(1, D), g.reshape(1, D), beta.reshape(1, D))


def _node_mlp_body(x_ref, a0_ref, a1_ref, W0_ref, b0_ref, W1_ref, b1_ref,
                   W2_ref, b2_ref, g_ref, beta_ref, out_ref):
    D = x_ref.shape[1]
    x = x_ref[...]
    aggr = a0_ref[...] + a1_ref[...]
    h = (jnp.dot(x, W0_ref[0:D, :], preferred_element_type=jnp.float32)
         + jnp.dot(aggr, W0_ref[D:2 * D, :], preferred_element_type=jnp.float32)
         + b0_ref[...])
    h = jnp.maximum(h, 0.0)
    h = jnp.maximum(jnp.dot(h, W1_ref[...], preferred_element_type=jnp.float32)
                    + b1_ref[...], 0.0)
    h = jnp.dot(h, W2_ref[...], preferred_element_type=jnp.float32) + b2_ref[...]
    mu = jnp.mean(h, axis=-1, keepdims=True)
    hc = h - mu
    var = jnp.mean(hc * hc, axis=-1, keepdims=True)
    hn = hc * lax.rsqrt(var + 1e-5)
    out_ref[...] = hn * g_ref[...] + beta_ref[...] + x


def _tc_node_mlp(x, a0, a1, W0, b0, W1, b1, W2, b2, g, beta, block):
    N, D = x.shape
    grid = (N // block,)
    row_spec = pl.BlockSpec((block, D), lambda i: (i, 0))
    full = lambda shape: pl.BlockSpec(shape, lambda i: (0,) * len(shape))
    return pl.pallas_call(
        _node_mlp_body,
        grid=grid,
        in_specs=[row_spec, row_spec, row_spec,
                  full((2 * D, D)), full((1, D)),
                  full((D, D)), full((1, D)),
                  full((D, D)), full((1, D)),
                  full((1, D)), full((1, D))],
        out_specs=row_spec,
        out_shape=jax.ShapeDtypeStruct((N, D), jnp.float32),
    )(x, a0, a1, W0, b0.reshape(1, D), W1, b1.reshape(1, D),
      W2, b2.reshape(1, D), g.reshape(1, D), beta.reshape(1, D))


def kernel(x, edge_index, edge_attr, mW0, mb0, mW1, mb1, mW2, mb2, mg, mbeta,
           uW0, ub0, uW1, ub1, uW2, ub2, ug, ubeta):
    N, D = x.shape
    E = edge_attr.shape[0]
    src = edge_index[0]
    dst = edge_index[1]

    xj, xi = _make_sc_gather(N, E, D, ch=400)(x, src, dst)
    new_ea, ea_out = _tc_edge_mlp(xi, xj, edge_attr,
                                  mW0, mb0, mW1, mb1, mW2, mb2, mg, mbeta,
                                  block=2000)
    NS = plsc.get_sparse_core_info().num_subcores
    Np = ((N + 8 * NS - 1) // (8 * NS)) * (8 * NS)
    zeros = jnp.zeros((Np, D), jnp.float32)
    partials = _make_sc_scatter(Np, E, D, ch=200)(new_ea, dst, zeros)
    x_out = _tc_node_mlp(x, partials[0, :N], partials[1, :N],
                         uW0, ub0, uW1, ub1, uW2, ub2, ug, ubeta,
                         block=1000)
    return (x_out, ea_out)
